# Initial kernel scaffold; baseline (speedup 1.0000x reference)
import jax
import jax.numpy as jnp
from jax.experimental import pallas as pl


def kernel(x, edge_index, W_in, b_in, g_in, be_in, W0, b0, g0, be0, W1, b1, g1, be1, W2, b2, W_out, b_out):
    raise NotImplementedError("write your pallas kernel here")


# speedup vs baseline: 8.7540x; 8.7540x over previous
import functools

import jax
import jax.numpy as jnp
from jax import lax
from jax.experimental import pallas as pl
from jax.experimental.pallas import tpu as pltpu
from jax.experimental.pallas import tpu_sc as plsc

N = 10000
E = 320000
D = 128
C = 64
EPS = 1e-5

NC = 2
NS = 16
NW = NC * NS

CH = 128
EPT = ((E // NW + CH - 1) // CH) * CH
E_PAD = EPT * NW
NCH = EPT // CH
RPT = 632
N_ACC = NS * RPT
DW = 16


@functools.cache
def _sc_kernels():
    mesh = plsc.VectorSubcoreMesh(
        core_axis_name="c", subcore_axis_name="s", num_cores=NC, num_subcores=NS)

    @functools.partial(
        pl.kernel,
        out_type=jax.ShapeDtypeStruct((NC * N_ACC,), jnp.float32),
        mesh=mesh,
        scratch_types=[
            pltpu.VMEM((CH,), jnp.int32),
            pltpu.VMEM((CH,), jnp.float32),
            pltpu.VMEM((RPT,), jnp.float32),
            pltpu.VMEM_SHARED((N_ACC,), jnp.float32),
        ],
    )
    def deg_sc(dst_hbm, ones_hbm, zrow_hbm, out_hbm, dst_v, ones_v, stage_v, acc_sh):
        c = lax.axis_index("c")
        s = lax.axis_index("s")
        w = c * NS + s
        pltpu.sync_copy(ones_hbm, ones_v)
        pltpu.sync_copy(zrow_hbm, stage_v)
        pltpu.sync_copy(stage_v, acc_sh.at[pl.ds(s * RPT, RPT)])
        plsc.subcore_barrier()
        base = w * EPT

        def body(k, carry):
            pltpu.sync_copy(dst_hbm.at[pl.ds(base + k * CH, CH)], dst_v)
            pltpu.sync_copy(ones_v, acc_sh.at[dst_v], add=True)
            return carry

        lax.fori_loop(0, NCH, body, 0)
        plsc.subcore_barrier()
        pltpu.sync_copy(acc_sh.at[pl.ds(s * RPT, RPT)], stage_v)
        pltpu.sync_copy(stage_v, out_hbm.at[pl.ds(c * N_ACC + s * RPT, RPT)])

    @functools.partial(
        pl.kernel,
        out_type=jax.ShapeDtypeStruct((NC * N_ACC, D), jnp.float32),
        mesh=mesh,
        scratch_types=[
            pltpu.VMEM((CH,), jnp.int32),
            pltpu.VMEM((CH,), jnp.int32),
            pltpu.VMEM((CH, D), jnp.float32),
            pltpu.VMEM((8, D), jnp.float32),
            pltpu.VMEM_SHARED((N_ACC, D), jnp.float32),
            pltpu.SemaphoreType.DMA,
        ],
    )
    def prop_sc(z_hbm, src_hbm, dst_hbm, zrows_hbm, out_hbm,
                src_v, dst_v, rows_v, stage_v, acc_sh, sem):
        c = lax.axis_index("c")
        s = lax.axis_index("s")
        w = c * NS + s
        pltpu.sync_copy(zrows_hbm, stage_v)

        def zbody(j, carry):
            pltpu.sync_copy(stage_v, acc_sh.at[pl.ds(s * RPT + j * 8, 8)])
            return carry

        lax.fori_loop(0, RPT // 8, zbody, 0)
        plsc.subcore_barrier()
        base = w * EPT

        def body(k, carry):
            off = base + k * CH
            pltpu.sync_copy(src_hbm.at[pl.ds(off, CH)], src_v)
            pltpu.sync_copy(dst_hbm.at[pl.ds(off, CH)], dst_v)
            pltpu.async_copy(z_hbm.at[src_v], rows_v, sem).wait()
            pltpu.sync_copy(rows_v, acc_sh.at[dst_v], add=True)
            return carry

        lax.fori_loop(0, NCH, body, 0)
        plsc.subcore_barrier()
        pltpu.sync_copy(
            acc_sh.at[pl.ds(s * RPT, RPT)],
            out_hbm.at[pl.ds(c * N_ACC + s * RPT, RPT)],
        )

    return deg_sc, prop_sc


def _bn_relu(h, g, be):
    mu = jnp.mean(h, axis=0, keepdims=True)
    var = jnp.mean((h - mu) * (h - mu), axis=0, keepdims=True)
    return jnp.maximum((h - mu) * lax.rsqrt(var + EPS) * g + be, 0.0)


def _tc_in_body(x_ref, w_ref, b_ref, g_ref, be_ref, w0_ref, d0_ref, d1_ref,
                dinv_ref, z0_ref):
    h = jnp.dot(x_ref[...], w_ref[...], preferred_element_type=jnp.float32)
    h = _bn_relu(h + b_ref[...], g_ref[...], be_ref[...])
    dinv = lax.rsqrt(d0_ref[...] + d1_ref[...] + 1.0)
    dinv_ref[...] = dinv
    z0_ref[...] = jnp.dot(h, w0_ref[...], preferred_element_type=jnp.float32) * dinv


def _tc_mid_body(p_ref, z_ref, dinv_ref, b_ref, g_ref, be_ref, w_ref,
                 zn_ref):
    dinv = dinv_ref[...]
    acc = p_ref[:N, :] + p_ref[N_ACC:N_ACC + N, :] + z_ref[...]
    h = _bn_relu(acc * dinv + b_ref[...], g_ref[...], be_ref[...])
    zn_ref[...] = jnp.dot(h, w_ref[...], preferred_element_type=jnp.float32) * dinv


def _tc_out_body(p_ref, z_ref, dinv_ref, b_ref, wout_ref, bout_ref, y_ref):
    acc = p_ref[:N, :] + p_ref[N_ACC:N_ACC + N, :] + z_ref[...]
    h = acc * dinv_ref[...] + b_ref[...]
    y_ref[...] = jnp.dot(h, wout_ref[...], preferred_element_type=jnp.float32) + bout_ref[...]


_f32 = jnp.float32

_tc_in = pl.pallas_call(
    _tc_in_body,
    out_shape=(jax.ShapeDtypeStruct((N, 1), _f32),
               jax.ShapeDtypeStruct((N, D), _f32)),
)

_tc_mid = pl.pallas_call(
    _tc_mid_body,
    out_shape=jax.ShapeDtypeStruct((N, D), _f32),
)

_tc_out = pl.pallas_call(
    _tc_out_body,
    out_shape=jax.ShapeDtypeStruct((N, C), _f32),
)


def kernel(x, edge_index, W_in, b_in, g_in, be_in, W0, b0, g0, be0,
           W1, b1, g1, be1, W2, b2, W_out, b_out):
    pad = E_PAD - E
    srcp = jnp.concatenate([edge_index[0], jnp.zeros((pad,), edge_index.dtype)])
    dstp = jnp.concatenate([edge_index[1], jnp.full((pad,), N, edge_index.dtype)])

    ones1 = jnp.ones((CH,), _f32)
    z1d = jnp.zeros((RPT,), _f32)
    zrows = jnp.zeros((8, D), _f32)

    _deg_sc, _prop_sc = _sc_kernels()
    degf = _deg_sc(dstp, ones1, z1d)
    d0 = degf[:N].reshape(N, 1)
    d1 = degf[N_ACC:N_ACC + N].reshape(N, 1)

    dinv, z0 = _tc_in(x, W_in, b_in.reshape(1, D), g_in.reshape(1, D),
                      be_in.reshape(1, D), W0, d0, d1)

    p0 = _prop_sc(z0, srcp, dstp, zrows)
    z1 = _tc_mid(p0, z0, dinv, b0.reshape(1, D), g0.reshape(1, D),
                 be0.reshape(1, D), W1)
    p1 = _prop_sc(z1, srcp, dstp, zrows)
    z2 = _tc_mid(p1, z1, dinv, b1.reshape(1, D), g1.reshape(1, D),
                 be1.reshape(1, D), W2)
    p2 = _prop_sc(z2, srcp, dstp, zrows)
    return _tc_out(p2, z2, dinv, b2.reshape(1, D), W_out, b_out.reshape(1, C))
